# baseline (speedup 1.0000x reference)
import functools

import jax
import jax.numpy as jnp
from jax import lax
from jax.experimental import pallas as pl
from jax.experimental.pallas import tpu as pltpu
from jax.experimental.pallas import tpu_sc as plsc

N = 10000
E = 320000
D = 128

NC = 2
NS = 16
NW = NC * NS

K = 128
CH = 80
EW = CH * K
E_PAD = NW * EW
ROWS_PAD = E_PAD // K
TRASH = N
ACC_ROWS = 10240
RPT = ACC_ROWS // NS
GRP = 8
CH0 = 144
CH1 = CH * 2 - CH0
PAIR = CH0 + CH1

_mesh = plsc.VectorSubcoreMesh(core_axis_name="c", subcore_axis_name="s")


@functools.partial(
    pl.kernel,
    out_type=jax.ShapeDtypeStruct((NC * ACC_ROWS,), jnp.float32),
    mesh=_mesh,
    scratch_types=[
        pltpu.VMEM((CH, K), jnp.int32),
        pltpu.VMEM((RPT,), jnp.float32),
        pltpu.VMEM((K,), jnp.float32),
        pltpu.VMEM_SHARED((ACC_ROWS,), jnp.float32),
    ],
)
def _k1_deg(dst_hbm, out_hbm, idx_v, zbuf, ones_v, acc):
    c = lax.axis_index("c")
    s = lax.axis_index("s")
    wid = s * NC + c
    for i in range(RPT // 16):
        zbuf[pl.ds(i * 16, 16)] = jnp.zeros((16,), jnp.float32)
    for i in range(K // 16):
        ones_v[pl.ds(i * 16, 16)] = jnp.ones((16,), jnp.float32)
    pltpu.sync_copy(zbuf, acc.at[pl.ds(s * RPT, RPT)])
    plsc.subcore_barrier()
    pltpu.sync_copy(dst_hbm.at[pl.ds(wid * CH, CH)], idx_v)

    def body(j, carry):
        pltpu.sync_copy(ones_v, acc.at[idx_v.at[j]], add=True)
        return carry

    lax.fori_loop(0, CH, body, 0)
    plsc.subcore_barrier()
    pltpu.sync_copy(acc.at[pl.ds(s * RPT, RPT)],
                    out_hbm.at[pl.ds(c * ACC_ROWS + s * RPT, RPT)])


def _k2_body(x_ref, degt_ref, w_ref, g_ref):
    deg = degt_ref[:, 0:1] + degt_ref[:, 1:2] + 1.0
    dinv = lax.rsqrt(deg)
    h = jnp.dot(x_ref[...], w_ref[...], preferred_element_type=jnp.float32)
    g_ref[...] = h * dinv


def _k2_scale(x, deg_t, w):
    blk = 1000
    return pl.pallas_call(
        _k2_body,
        grid=(N // blk,),
        in_specs=[
            pl.BlockSpec((blk, D), lambda i: (i, 0)),
            pl.BlockSpec((blk, 2), lambda i: (i, 0)),
            pl.BlockSpec((D, D), lambda i: (0, 0)),
        ],
        out_specs=pl.BlockSpec((blk, D), lambda i: (i, 0)),
        out_shape=jax.ShapeDtypeStruct((N, D), jnp.float32),
    )(x, deg_t, w)


@functools.partial(
    pl.kernel,
    out_type=jax.ShapeDtypeStruct((NC, ACC_ROWS, D), jnp.float32),
    mesh=_mesh,
    scratch_types=[
        pltpu.VMEM((GRP, K), jnp.int32),
        pltpu.VMEM((GRP, K), jnp.int32),
        pltpu.VMEM((K, D), jnp.float32),
        pltpu.VMEM((K, D), jnp.float32),
        pltpu.VMEM((16, D), jnp.float32),
        pltpu.VMEM_SHARED((ACC_ROWS, D), jnp.float32),
        pltpu.SemaphoreType.DMA,
    ],
)
def _k3_edges(g_hbm, src_hbm, dst_hbm, out_hbm,
              idx_s, idx_d, rows0, rows1, zbuf, acc, sem):
    c = lax.axis_index("c")
    s = lax.axis_index("s")
    wid = s * NC + c
    for i in range(16):
        for j in range(D // 16):
            zbuf[i, pl.ds(j * 16, 16)] = jnp.zeros((16,), jnp.float32)
    for i in range(RPT // 16):
        pltpu.async_copy(zbuf, acc.at[pl.ds(s * RPT + i * 16, 16)], sem)
    for i in range(RPT // 16):
        pltpu.make_async_copy(zbuf, acc.at[pl.ds(s * RPT + i * 16, 16)],
                              sem).wait()
    plsc.subcore_barrier()

    mych = jnp.where(c == 0, jnp.int32(CH0), jnp.int32(CH1))
    rbase = s * PAIR + c * CH0

    def group(gi, carry):
        base = pl.multiple_of(rbase + gi * GRP, GRP)
        pltpu.sync_copy(src_hbm.at[pl.ds(base, GRP)], idx_s)
        pltpu.sync_copy(dst_hbm.at[pl.ds(base, GRP)], idx_d)
        pltpu.async_copy(g_hbm.at[idx_s.at[0]], rows0, sem)

        def pair(t, carry2):
            j = t * 2
            pltpu.make_async_copy(g_hbm.at[idx_s.at[j]], rows0, sem).wait()
            pltpu.async_copy(g_hbm.at[idx_s.at[j + 1]], rows1, sem)
            pltpu.sync_copy(rows0, acc.at[idx_d.at[j]], add=True)
            pltpu.make_async_copy(g_hbm.at[idx_s.at[j + 1]], rows1,
                                  sem).wait()

            @pl.when(j + 2 < GRP)
            def _():
                pltpu.async_copy(g_hbm.at[idx_s.at[j + 2]], rows0, sem)

            pltpu.sync_copy(rows1, acc.at[idx_d.at[j + 1]], add=True)
            return carry2

        lax.fori_loop(0, GRP // 2, pair, 0)
        return carry

    lax.fori_loop(0, mych // GRP, group, 0)
    plsc.subcore_barrier()
    pltpu.sync_copy(acc.at[pl.ds(s * RPT, RPT)],
                    out_hbm.at[c, pl.ds(s * RPT, RPT), :])


def _k4_body(part_ref, g_ref, degt_ref, b_ref, out_ref):
    deg = degt_ref[:, 0:1] + degt_ref[:, 1:2] + 1.0
    dinv = lax.rsqrt(deg)
    a = part_ref[0] + part_ref[1] + g_ref[...]
    out_ref[...] = jnp.maximum(a * dinv + b_ref[...], 0.0)


def _k4_combine(parts, g, deg_t, b2d):
    blk = 1000
    return pl.pallas_call(
        _k4_body,
        grid=(N // blk,),
        in_specs=[
            pl.BlockSpec((NC, blk, D), lambda i: (0, i, 0)),
            pl.BlockSpec((blk, D), lambda i: (i, 0)),
            pl.BlockSpec((blk, 2), lambda i: (i, 0)),
            pl.BlockSpec((1, D), lambda i: (0, 0)),
        ],
        out_specs=pl.BlockSpec((blk, D), lambda i: (i, 0)),
        out_shape=jax.ShapeDtypeStruct((N, D), jnp.float32),
    )(parts, g, deg_t, b2d)


def kernel(x, edge_index, W, b):
    ei = edge_index.astype(jnp.int32)
    pad = E_PAD - E
    src = jnp.concatenate([ei[0], jnp.zeros((pad,), jnp.int32)])
    dst = jnp.concatenate([ei[1], jnp.full((pad,), TRASH, jnp.int32)])
    src2d = src.reshape(ROWS_PAD, K)
    dst2d = dst.reshape(ROWS_PAD, K)

    deg_parts = _k1_deg(dst2d).reshape(NC, ACC_ROWS)
    deg_t = jnp.transpose(deg_parts)[:N]
    g = _k2_scale(x, deg_t, W)
    parts = _k3_edges(g, src2d, dst2d)
    b2d = b.reshape(1, D)
    return _k4_combine(parts, g, deg_t, b2d)


# speedup vs baseline: 1.1412x; 1.1412x over previous
import functools

import jax
import jax.numpy as jnp
from jax import lax
from jax.experimental import pallas as pl
from jax.experimental.pallas import tpu as pltpu
from jax.experimental.pallas import tpu_sc as plsc

N = 10000
E = 320000
D = 128

NC = 2
NS = 16
NW = NC * NS

K = 128
CH = 80
EW = CH * K
E_PAD = NW * EW
ROWS_PAD = E_PAD // K
TRASH = N
ACC_ROWS = 10240
RPT = ACC_ROWS // NS
GRP = 8
CH0 = 112
CH1 = CH * 2 - CH0
PAIR = CH0 + CH1

_mesh = plsc.VectorSubcoreMesh(core_axis_name="c", subcore_axis_name="s")


@functools.partial(
    pl.kernel,
    out_type=jax.ShapeDtypeStruct((NC * ACC_ROWS,), jnp.float32),
    mesh=_mesh,
    scratch_types=[
        pltpu.VMEM((CH, K), jnp.int32),
        pltpu.VMEM((RPT,), jnp.float32),
        pltpu.VMEM((K,), jnp.float32),
        pltpu.VMEM_SHARED((ACC_ROWS,), jnp.float32),
    ],
)
def _k1_deg(dst_hbm, out_hbm, idx_v, zbuf, ones_v, acc):
    c = lax.axis_index("c")
    s = lax.axis_index("s")
    wid = s * NC + c
    for i in range(RPT // 16):
        zbuf[pl.ds(i * 16, 16)] = jnp.zeros((16,), jnp.float32)
    for i in range(K // 16):
        ones_v[pl.ds(i * 16, 16)] = jnp.ones((16,), jnp.float32)
    pltpu.sync_copy(zbuf, acc.at[pl.ds(s * RPT, RPT)])
    plsc.subcore_barrier()
    pltpu.sync_copy(dst_hbm.at[pl.ds(wid * CH, CH)], idx_v)

    def body(j, carry):
        pltpu.sync_copy(ones_v, acc.at[idx_v.at[j]], add=True)
        return carry

    lax.fori_loop(0, CH, body, 0)
    plsc.subcore_barrier()
    pltpu.sync_copy(acc.at[pl.ds(s * RPT, RPT)],
                    out_hbm.at[pl.ds(c * ACC_ROWS + s * RPT, RPT)])


def _k2_body(x_ref, degt_ref, w_ref, g_ref):
    deg = degt_ref[:, 0:1] + degt_ref[:, 1:2] + 1.0
    dinv = lax.rsqrt(deg)
    h = jnp.dot(x_ref[...], w_ref[...], preferred_element_type=jnp.float32)
    g_ref[...] = h * dinv


def _k2_scale(x, deg_t, w):
    blk = 1000
    return pl.pallas_call(
        _k2_body,
        grid=(N // blk,),
        in_specs=[
            pl.BlockSpec((blk, D), lambda i: (i, 0)),
            pl.BlockSpec((blk, 2), lambda i: (i, 0)),
            pl.BlockSpec((D, D), lambda i: (0, 0)),
        ],
        out_specs=pl.BlockSpec((blk, D), lambda i: (i, 0)),
        out_shape=jax.ShapeDtypeStruct((N, D), jnp.float32),
    )(x, deg_t, w)


@functools.partial(
    pl.kernel,
    out_type=jax.ShapeDtypeStruct((NC, ACC_ROWS, D), jnp.float32),
    mesh=_mesh,
    scratch_types=[
        pltpu.VMEM((GRP, K), jnp.int32),
        pltpu.VMEM((GRP, K), jnp.int32),
        pltpu.VMEM((K, D), jnp.float32),
        pltpu.VMEM((K, D), jnp.float32),
        pltpu.VMEM((16, D), jnp.float32),
        pltpu.VMEM_SHARED((ACC_ROWS, D), jnp.float32),
        pltpu.SemaphoreType.DMA,
    ],
)
def _k3_edges(g_hbm, src_hbm, dst_hbm, out_hbm,
              idx_s, idx_d, rows0, rows1, zbuf, acc, sem):
    c = lax.axis_index("c")
    s = lax.axis_index("s")
    wid = s * NC + c
    for i in range(16):
        for j in range(D // 16):
            zbuf[i, pl.ds(j * 16, 16)] = jnp.zeros((16,), jnp.float32)
    for i in range(RPT // 16):
        pltpu.async_copy(zbuf, acc.at[pl.ds(s * RPT + i * 16, 16)], sem)
    for i in range(RPT // 16):
        pltpu.make_async_copy(zbuf, acc.at[pl.ds(s * RPT + i * 16, 16)],
                              sem).wait()
    plsc.subcore_barrier()

    mych = jnp.where(c == 0, jnp.int32(CH0), jnp.int32(CH1))
    rbase = s * PAIR + c * CH0

    def group(gi, carry):
        base = pl.multiple_of(rbase + gi * GRP, GRP)
        pltpu.sync_copy(src_hbm.at[pl.ds(base, GRP)], idx_s)
        pltpu.sync_copy(dst_hbm.at[pl.ds(base, GRP)], idx_d)
        pltpu.async_copy(g_hbm.at[idx_s.at[0]], rows0, sem)

        def pair(t, carry2):
            j = t * 2
            pltpu.make_async_copy(g_hbm.at[idx_s.at[j]], rows0, sem).wait()
            pltpu.async_copy(g_hbm.at[idx_s.at[j + 1]], rows1, sem)
            pltpu.sync_copy(rows0, acc.at[idx_d.at[j]], add=True)
            pltpu.make_async_copy(g_hbm.at[idx_s.at[j + 1]], rows1,
                                  sem).wait()

            @pl.when(j + 2 < GRP)
            def _():
                pltpu.async_copy(g_hbm.at[idx_s.at[j + 2]], rows0, sem)

            pltpu.sync_copy(rows1, acc.at[idx_d.at[j + 1]], add=True)
            return carry2

        lax.fori_loop(0, GRP // 2, pair, 0)
        return carry

    lax.fori_loop(0, mych // GRP, group, 0)
    plsc.subcore_barrier()
    pltpu.sync_copy(acc.at[pl.ds(s * RPT, RPT)],
                    out_hbm.at[c, pl.ds(s * RPT, RPT), :])


def _k4_body(part_ref, g_ref, degt_ref, b_ref, out_ref):
    deg = degt_ref[:, 0:1] + degt_ref[:, 1:2] + 1.0
    dinv = lax.rsqrt(deg)
    a = part_ref[0] + part_ref[1] + g_ref[...]
    out_ref[...] = jnp.maximum(a * dinv + b_ref[...], 0.0)


def _k4_combine(parts, g, deg_t, b2d):
    blk = 1000
    return pl.pallas_call(
        _k4_body,
        grid=(N // blk,),
        in_specs=[
            pl.BlockSpec((NC, blk, D), lambda i: (0, i, 0)),
            pl.BlockSpec((blk, D), lambda i: (i, 0)),
            pl.BlockSpec((blk, 2), lambda i: (i, 0)),
            pl.BlockSpec((1, D), lambda i: (0, 0)),
        ],
        out_specs=pl.BlockSpec((blk, D), lambda i: (i, 0)),
        out_shape=jax.ShapeDtypeStruct((N, D), jnp.float32),
    )(parts, g, deg_t, b2d)


def kernel(x, edge_index, W, b):
    ei = edge_index.astype(jnp.int32)
    pad = E_PAD - E
    src = jnp.concatenate([ei[0], jnp.zeros((pad,), jnp.int32)])
    dst = jnp.concatenate([ei[1], jnp.full((pad,), TRASH, jnp.int32)])
    src2d = src.reshape(ROWS_PAD, K)
    dst2d = dst.reshape(ROWS_PAD, K)

    deg_parts = _k1_deg(dst2d).reshape(NC, ACC_ROWS)
    deg_t = jnp.transpose(deg_parts)[:N]
    g = _k2_scale(x, deg_t, W)
    parts = _k3_edges(g, src2d, dst2d)
    b2d = b.reshape(1, D)
    return _k4_combine(parts, g, deg_t, b2d)


# speedup vs baseline: 1.2269x; 1.0751x over previous
import functools

import jax
import jax.numpy as jnp
from jax import lax
from jax.experimental import pallas as pl
from jax.experimental.pallas import tpu as pltpu
from jax.experimental.pallas import tpu_sc as plsc

N = 10000
E = 320000
D = 128

NC = 2
NS = 16
NW = NC * NS

K = 128
CH = 80
EW = CH * K
E_PAD = NW * EW
ROWS_PAD = E_PAD // K
TRASH = N
ACC_ROWS = 10240
RPT = ACC_ROWS // NS
GRP = 8
CH0 = 120
CH1 = CH * 2 - CH0
PAIR = CH0 + CH1

_mesh = plsc.VectorSubcoreMesh(core_axis_name="c", subcore_axis_name="s")


@functools.partial(
    pl.kernel,
    out_type=jax.ShapeDtypeStruct((NC * ACC_ROWS,), jnp.float32),
    mesh=_mesh,
    scratch_types=[
        pltpu.VMEM((CH, K), jnp.int32),
        pltpu.VMEM((RPT,), jnp.float32),
        pltpu.VMEM((K,), jnp.float32),
        pltpu.VMEM_SHARED((ACC_ROWS,), jnp.float32),
    ],
)
def _k1_deg(dst_hbm, out_hbm, idx_v, zbuf, ones_v, acc):
    c = lax.axis_index("c")
    s = lax.axis_index("s")
    wid = s * NC + c
    for i in range(RPT // 16):
        zbuf[pl.ds(i * 16, 16)] = jnp.zeros((16,), jnp.float32)
    for i in range(K // 16):
        ones_v[pl.ds(i * 16, 16)] = jnp.ones((16,), jnp.float32)
    pltpu.sync_copy(zbuf, acc.at[pl.ds(s * RPT, RPT)])
    plsc.subcore_barrier()
    pltpu.sync_copy(dst_hbm.at[pl.ds(wid * CH, CH)], idx_v)

    def body(j, carry):
        pltpu.sync_copy(ones_v, acc.at[idx_v.at[j]], add=True)
        return carry

    lax.fori_loop(0, CH, body, 0)
    plsc.subcore_barrier()
    pltpu.sync_copy(acc.at[pl.ds(s * RPT, RPT)],
                    out_hbm.at[pl.ds(c * ACC_ROWS + s * RPT, RPT)])


def _k2_body(x_ref, degt_ref, w_ref, g_ref):
    deg = degt_ref[:, 0:1] + degt_ref[:, 1:2] + 1.0
    dinv = lax.rsqrt(deg)
    h = jnp.dot(x_ref[...], w_ref[...], preferred_element_type=jnp.float32)
    g_ref[...] = h * dinv


def _k2_scale(x, deg_t, w):
    blk = 1000
    return pl.pallas_call(
        _k2_body,
        grid=(N // blk,),
        in_specs=[
            pl.BlockSpec((blk, D), lambda i: (i, 0)),
            pl.BlockSpec((blk, 2), lambda i: (i, 0)),
            pl.BlockSpec((D, D), lambda i: (0, 0)),
        ],
        out_specs=pl.BlockSpec((blk, D), lambda i: (i, 0)),
        out_shape=jax.ShapeDtypeStruct((N, D), jnp.float32),
    )(x, deg_t, w)


@functools.partial(
    pl.kernel,
    out_type=jax.ShapeDtypeStruct((NC, ACC_ROWS, D), jnp.float32),
    mesh=_mesh,
    scratch_types=[
        pltpu.VMEM((GRP, K), jnp.int32),
        pltpu.VMEM((GRP, K), jnp.int32),
        pltpu.VMEM((K, D), jnp.float32),
        pltpu.VMEM((K, D), jnp.float32),
        pltpu.VMEM((16, D), jnp.float32),
        pltpu.VMEM_SHARED((ACC_ROWS, D), jnp.float32),
        pltpu.SemaphoreType.DMA,
    ],
)
def _k3_edges(g_hbm, src_hbm, dst_hbm, out_hbm,
              idx_s, idx_d, rows0, rows1, zbuf, acc, sem):
    c = lax.axis_index("c")
    s = lax.axis_index("s")
    wid = s * NC + c
    for i in range(16):
        for j in range(D // 16):
            zbuf[i, pl.ds(j * 16, 16)] = jnp.zeros((16,), jnp.float32)
    for i in range(RPT // 16):
        pltpu.async_copy(zbuf, acc.at[pl.ds(s * RPT + i * 16, 16)], sem)
    for i in range(RPT // 16):
        pltpu.make_async_copy(zbuf, acc.at[pl.ds(s * RPT + i * 16, 16)],
                              sem).wait()
    plsc.subcore_barrier()

    mych = jnp.where(c == 0, jnp.int32(CH0), jnp.int32(CH1))
    rbase = s * PAIR + c * CH0

    def group(gi, carry):
        base = pl.multiple_of(rbase + gi * GRP, GRP)
        pltpu.sync_copy(src_hbm.at[pl.ds(base, GRP)], idx_s)
        pltpu.sync_copy(dst_hbm.at[pl.ds(base, GRP)], idx_d)
        pltpu.async_copy(g_hbm.at[idx_s.at[0]], rows0, sem)

        def pair(t, carry2):
            j = t * 2
            pltpu.make_async_copy(g_hbm.at[idx_s.at[j]], rows0, sem).wait()
            pltpu.async_copy(g_hbm.at[idx_s.at[j + 1]], rows1, sem)
            pltpu.sync_copy(rows0, acc.at[idx_d.at[j]], add=True)
            pltpu.make_async_copy(g_hbm.at[idx_s.at[j + 1]], rows1,
                                  sem).wait()

            @pl.when(j + 2 < GRP)
            def _():
                pltpu.async_copy(g_hbm.at[idx_s.at[j + 2]], rows0, sem)

            pltpu.sync_copy(rows1, acc.at[idx_d.at[j + 1]], add=True)
            return carry2

        lax.fori_loop(0, GRP // 2, pair, 0)
        return carry

    lax.fori_loop(0, mych // GRP, group, 0)
    plsc.subcore_barrier()
    pltpu.sync_copy(acc.at[pl.ds(s * RPT, RPT)],
                    out_hbm.at[c, pl.ds(s * RPT, RPT), :])


def _k4_body(part_ref, g_ref, degt_ref, b_ref, out_ref):
    deg = degt_ref[:, 0:1] + degt_ref[:, 1:2] + 1.0
    dinv = lax.rsqrt(deg)
    a = part_ref[0] + part_ref[1] + g_ref[...]
    out_ref[...] = jnp.maximum(a * dinv + b_ref[...], 0.0)


def _k4_combine(parts, g, deg_t, b2d):
    blk = 1000
    return pl.pallas_call(
        _k4_body,
        grid=(N // blk,),
        in_specs=[
            pl.BlockSpec((NC, blk, D), lambda i: (0, i, 0)),
            pl.BlockSpec((blk, D), lambda i: (i, 0)),
            pl.BlockSpec((blk, 2), lambda i: (i, 0)),
            pl.BlockSpec((1, D), lambda i: (0, 0)),
        ],
        out_specs=pl.BlockSpec((blk, D), lambda i: (i, 0)),
        out_shape=jax.ShapeDtypeStruct((N, D), jnp.float32),
    )(parts, g, deg_t, b2d)


def kernel(x, edge_index, W, b):
    ei = edge_index.astype(jnp.int32)
    pad = E_PAD - E
    src = jnp.concatenate([ei[0], jnp.zeros((pad,), jnp.int32)])
    dst = jnp.concatenate([ei[1], jnp.full((pad,), TRASH, jnp.int32)])
    src2d = src.reshape(ROWS_PAD, K)
    dst2d = dst.reshape(ROWS_PAD, K)

    deg_parts = _k1_deg(dst2d).reshape(NC, ACC_ROWS)
    deg_t = jnp.transpose(deg_parts)[:N]
    g = _k2_scale(x, deg_t, W)
    parts = _k3_edges(g, src2d, dst2d)
    b2d = b.reshape(1, D)
    return _k4_combine(parts, g, deg_t, b2d)
